# Initial kernel scaffold; baseline (speedup 1.0000x reference)
import jax
import jax.numpy as jnp
from jax.experimental import pallas as pl


def kernel(all_emb, edge_index, edge_weight):
    raise NotImplementedError("write your pallas kernel here")


# speedup vs baseline: 4.6974x; 4.6974x over previous
import functools

import jax
import jax.numpy as jnp
from jax import lax
from jax.experimental import pallas as pl
from jax.experimental.pallas import tpu as pltpu
from jax.experimental.pallas import tpu_sc as plsc

N = 10000
NP = 10240
D = 128
E = 320000
HD = D // 2
NC = 2
NS = 16
R = NP // NS
CHUNK = 128
GC = 8
GROUPS = 20
NCHUNK = GC * GROUPS
EP = NS * NCHUNK * CHUNK


def _body(emb2, src3, dst3, wv3, zeros_h, out,
          spA, spB, rows, rows2, sidx_g, didx_g, w_g):
    cid = lax.axis_index("c")
    sid = lax.axis_index("s")
    row0 = sid * R
    rslice = pl.ds(row0, R)

    pltpu.sync_copy(emb2.at[cid, rslice], spA.at[rslice])
    pltpu.sync_copy(zeros_h.at[rslice], spB.at[rslice])
    plsc.subcore_barrier()

    def do_layer(src_sp, dst_sp):
        def group_body(gj, _):
            gsl = pl.ds(gj * GC, GC)
            pltpu.sync_copy(src3.at[sid, gsl], sidx_g)
            pltpu.sync_copy(dst3.at[sid, gsl], didx_g)
            pltpu.sync_copy(wv3.at[sid, gsl], w_g)
            for g2 in range(GC):
                pltpu.sync_copy(src_sp.at[sidx_g.at[g2]], rows)

                def scale_body(g, _):
                    wv = w_g[g2, pl.ds(g * 16, 16)]
                    for t in range(16):
                        w = wv[t]
                        i = g * 16 + t
                        for k in range(4):
                            sl = pl.ds(k * 16, 16)
                            rows[i, sl] = rows[i, sl] * w
                    return 0

                lax.fori_loop(0, CHUNK // 16, scale_body, 0)
                pltpu.sync_copy(rows, dst_sp.at[didx_g.at[g2]], add=True)
            return 0

        lax.fori_loop(0, GROUPS, group_body, 0)

    def accumulate(dst_sp, first, last):
        for c5 in range(R // CHUNK):
            sl_r = pl.ds(row0 + c5 * CHUNK, CHUNK)
            pltpu.sync_copy(dst_sp.at[sl_r], rows)
            if first:
                pltpu.sync_copy(emb2.at[cid, sl_r], rows2)
            else:
                pltpu.sync_copy(out.at[cid, sl_r], rows2)

            def add_body(i, _):
                for k in range(4):
                    sl = pl.ds(k * 16, 16)
                    v = rows2[i, sl] + rows[i, sl]
                    if last:
                        v = v * 0.25
                    rows2[i, sl] = v
                return 0

            lax.fori_loop(0, CHUNK, add_body, 0)
            pltpu.sync_copy(rows2, out.at[cid, sl_r])

    do_layer(spA, spB)
    plsc.subcore_barrier()
    accumulate(spB, first=True, last=False)
    pltpu.sync_copy(zeros_h.at[rslice], spA.at[rslice])
    plsc.subcore_barrier()

    do_layer(spB, spA)
    plsc.subcore_barrier()
    accumulate(spA, first=False, last=False)
    pltpu.sync_copy(zeros_h.at[rslice], spB.at[rslice])
    plsc.subcore_barrier()

    do_layer(spA, spB)
    plsc.subcore_barrier()
    accumulate(spB, first=False, last=True)


_sc_kernel = functools.partial(
    pl.kernel,
    out_type=jax.ShapeDtypeStruct((NC, NP, HD), jnp.float32),
    mesh=plsc.VectorSubcoreMesh(core_axis_name="c", subcore_axis_name="s"),
    compiler_params=pltpu.CompilerParams(use_tc_tiling_on_sc=False),
    scratch_types=[
        pltpu.VMEM_SHARED((NP, HD), jnp.float32),
        pltpu.VMEM_SHARED((NP, HD), jnp.float32),
        pltpu.VMEM((CHUNK, HD), jnp.float32),
        pltpu.VMEM((CHUNK, HD), jnp.float32),
        pltpu.VMEM((GC, CHUNK), jnp.int32),
        pltpu.VMEM((GC, CHUNK), jnp.int32),
        pltpu.VMEM((GC, CHUNK), jnp.float32),
    ],
)(_body)


@jax.jit
def kernel(all_emb, edge_index, edge_weight):
    src = edge_index[0]
    dst = edge_index[1]
    pad = EP - E
    src_p = jnp.pad(src, (0, pad)).reshape(NS, NCHUNK, CHUNK)
    dst_p = jnp.pad(dst, (0, pad)).reshape(NS, NCHUNK, CHUNK)
    w_p = jnp.pad(edge_weight, (0, pad)).reshape(NS, NCHUNK, CHUNK)
    emb_p = jnp.pad(all_emb, ((0, NP - N), (0, 0)))
    emb2 = jnp.stack([emb_p[:, :HD], emb_p[:, HD:]])
    zeros_h = jnp.zeros((NP, HD), jnp.float32)
    out = _sc_kernel(emb2, src_p, dst_p, w_p, zeros_h)
    return out[:, :N, :].transpose(1, 0, 2).reshape(N, D)


# speedup vs baseline: 5.4626x; 1.1629x over previous
import functools

import jax
import jax.numpy as jnp
from jax import lax
from jax.experimental import pallas as pl
from jax.experimental.pallas import tpu as pltpu
from jax.experimental.pallas import tpu_sc as plsc

N = 10000
NP = 10240
D = 128
E = 320000
HD = D // 2
NC = 2
NS = 16
R = NP // NS
CHUNK = 128
GC = 8
GROUPS = 20
NCHUNK = GC * GROUPS
EP = NS * NCHUNK * CHUNK


def _body(emb2, edata, zeros_h, out,
          spA, spB, rows0, rows1, edv, sg0, sg1, ss0, ss1):
    cid = lax.axis_index("c")
    sid = lax.axis_index("s")
    row0 = sid * R
    rslice = pl.ds(row0, R)
    rows = [rows0, rows1]
    sg = [sg0, sg1]
    ss = [ss0, ss1]

    pltpu.sync_copy(emb2.at[cid, rslice], spA.at[rslice])
    pltpu.sync_copy(zeros_h.at[rslice], spB.at[rslice])
    plsc.subcore_barrier()

    def scale(buf, g2):
        def scale_body(g, _):
            wv = plsc.bitcast(edv[g2, 2, pl.ds(g * 16, 16)], jnp.float32)
            for t in range(16):
                w = wv[t]
                i = g * 16 + t
                for k in range(4):
                    sl = pl.ds(k * 16, 16)
                    buf[i, sl] = buf[i, sl] * w
            return 0

        lax.fori_loop(0, CHUNK // 16, scale_body, 0)

    def do_layer(src_sp, dst_sp):
        def group_body(gj, _):
            pltpu.sync_copy(edata.at[sid, pl.ds(gj * GC, GC)], edv)
            gd = {}
            sd = {}
            for c in range(GC + 1):
                if c >= 2 and (c - 2) in sd:
                    sd[c - 2].wait()
                if c < GC:
                    b = c % 2
                    gd[c] = pltpu.async_copy(
                        src_sp.at[edv.at[c, 0]], rows[b], sg[b])
                if c >= 1:
                    p = c - 1
                    b = p % 2
                    gd[p].wait()
                    scale(rows[b], p)
                    sd[p] = pltpu.async_copy(
                        rows[b], dst_sp.at[edv.at[p, 1]], ss[b], add=True)
            sd[GC - 1].wait()
            return 0

        lax.fori_loop(0, GROUPS, group_body, 0)

    def accumulate(dst_sp, first, last):
        for c5 in range(R // CHUNK):
            sl_r = pl.ds(row0 + c5 * CHUNK, CHUNK)
            pltpu.sync_copy(dst_sp.at[sl_r], rows0)
            if first:
                pltpu.sync_copy(emb2.at[cid, sl_r], rows1)
            else:
                pltpu.sync_copy(out.at[cid, sl_r], rows1)

            def add_body(i, _):
                for k in range(4):
                    sl = pl.ds(k * 16, 16)
                    v = rows1[i, sl] + rows0[i, sl]
                    if last:
                        v = v * 0.25
                    rows1[i, sl] = v
                return 0

            lax.fori_loop(0, CHUNK, add_body, 0)
            pltpu.sync_copy(rows1, out.at[cid, sl_r])

    do_layer(spA, spB)
    plsc.subcore_barrier()
    accumulate(spB, first=True, last=False)
    pltpu.sync_copy(zeros_h.at[rslice], spA.at[rslice])
    plsc.subcore_barrier()

    do_layer(spB, spA)
    plsc.subcore_barrier()
    accumulate(spA, first=False, last=False)
    pltpu.sync_copy(zeros_h.at[rslice], spB.at[rslice])
    plsc.subcore_barrier()

    do_layer(spA, spB)
    plsc.subcore_barrier()
    accumulate(spB, first=False, last=True)


_sc_kernel = functools.partial(
    pl.kernel,
    out_type=jax.ShapeDtypeStruct((NC, NP, HD), jnp.float32),
    mesh=plsc.VectorSubcoreMesh(core_axis_name="c", subcore_axis_name="s"),
    compiler_params=pltpu.CompilerParams(use_tc_tiling_on_sc=False, needs_layout_passes=False),
    scratch_types=[
        pltpu.VMEM_SHARED((NP, HD), jnp.float32),
        pltpu.VMEM_SHARED((NP, HD), jnp.float32),
        pltpu.VMEM((CHUNK, HD), jnp.float32),
        pltpu.VMEM((CHUNK, HD), jnp.float32),
        pltpu.VMEM((GC, 3, CHUNK), jnp.int32),
        pltpu.SemaphoreType.DMA,
        pltpu.SemaphoreType.DMA,
        pltpu.SemaphoreType.DMA,
        pltpu.SemaphoreType.DMA,
    ],
)(_body)


@jax.jit
def kernel(all_emb, edge_index, edge_weight):
    src = edge_index[0]
    dst = edge_index[1]
    pad = EP - E
    src_p = jnp.pad(src, (0, pad)).reshape(NS, NCHUNK, CHUNK)
    dst_p = jnp.pad(dst, (0, pad)).reshape(NS, NCHUNK, CHUNK)
    w_p = jnp.pad(edge_weight, (0, pad)).reshape(NS, NCHUNK, CHUNK)
    edata = jnp.stack(
        [src_p, dst_p, jax.lax.bitcast_convert_type(w_p, jnp.int32)], axis=2)
    emb_p = jnp.pad(all_emb, ((0, NP - N), (0, 0)))
    emb2 = jnp.stack([emb_p[:, :HD], emb_p[:, HD:]])
    zeros_h = jnp.zeros((NP, HD), jnp.float32)
    out = _sc_kernel(emb2, edata, zeros_h)
    return out[:, :N, :].transpose(1, 0, 2).reshape(N, D)
